# baseline (speedup 1.0000x reference)
import jax
import jax.numpy as jnp
from jax import lax
from jax.experimental import pallas as pl
from jax.experimental.pallas import tpu as pltpu
from jax.experimental.pallas import tpu_sc as plsc

N = 10000
E = 320000
NCORES = 2
NSUB = 16
NW = NCORES * NSUB
BATCH = 128
EPT = 10240
NBATCH = EPT // BATCH
NBUF = 4
NGRP = NBATCH // NBUF
NPAD = 10240
ROWS_PT = NPAD // NSUB
DEGW = 8
FASTC = 0
GF = 28
GS = 12


def _sc_mesh():
    return plsc.VectorSubcoreMesh(core_axis_name="c", subcore_axis_name="s")


_SC_PARAMS = pltpu.CompilerParams(use_tc_tiling_on_sc=False)


def _deg_body(src_hbm, dst_hbm, ones_hbm, zeros_hbm, od_out, id_out,
              src_v, dst_v, ones_v, od_acc, id_acc, *sems):
    c = lax.axis_index("c")
    s = lax.axis_index("s")
    wid = s * NCORES + c
    base = s * ROWS_PT
    pltpu.sync_copy(ones_hbm, ones_v)
    pltpu.sync_copy(src_hbm.at[wid], src_v)
    pltpu.sync_copy(dst_hbm.at[wid], dst_v)
    pltpu.sync_copy(zeros_hbm.at[pl.ds(base, ROWS_PT)], od_acc.at[pl.ds(base, ROWS_PT)])
    pltpu.sync_copy(zeros_hbm.at[pl.ds(base, ROWS_PT)], id_acc.at[pl.ds(base, ROWS_PT)])
    plsc.subcore_barrier()

    def grp(g, carry):
        j = g * NBUF
        ds = []
        for b in range(NBUF):
            ds.append(pltpu.async_copy(ones_v, od_acc.at[src_v.at[j + b]],
                                       sems[b], add=True))
            ds.append(pltpu.async_copy(ones_v, id_acc.at[dst_v.at[j + b]],
                                       sems[NBUF + b], add=True))
        for d in ds:
            d.wait()
        return carry

    lax.fori_loop(0, NGRP, grp, 0)
    plsc.subcore_barrier()
    pltpu.sync_copy(od_acc.at[pl.ds(base, ROWS_PT)], od_out.at[c, pl.ds(base, ROWS_PT)])
    pltpu.sync_copy(id_acc.at[pl.ds(base, ROWS_PT)], id_out.at[c, pl.ds(base, ROWS_PT)])


def _degree_pass(src3, dst3, ones_deg, zeros_deg):
    f = pl.kernel(
        _deg_body,
        mesh=_sc_mesh(),
        compiler_params=_SC_PARAMS,
        out_type=[
            jax.ShapeDtypeStruct((NCORES, NPAD, DEGW), jnp.float32),
            jax.ShapeDtypeStruct((NCORES, NPAD, DEGW), jnp.float32),
        ],
        scratch_types=[
            pltpu.VMEM((NBATCH, BATCH), jnp.int32),
            pltpu.VMEM((NBATCH, BATCH), jnp.int32),
            pltpu.VMEM((BATCH, DEGW), jnp.float32),
            pltpu.VMEM_SHARED((NPAD, DEGW), jnp.float32),
            pltpu.VMEM_SHARED((NPAD, DEGW), jnp.float32),
        ] + [pltpu.SemaphoreType.DMA] * (2 * NBUF),
    )
    return f(src3, dst3, ones_deg, zeros_deg)


def _make_agg_body(width, batch, nbuf):
    def body(x_hbm, src_hbm, dst_hbm, zeros_hbm, out_hbm,
             src_v, dst_v, *rest):
        bufs = rest[:nbuf]
        acc = rest[nbuf]
        sems = rest[nbuf + 1:]
        c = lax.axis_index("c")
        s = lax.axis_index("s")
        wid = s * NCORES + c
        base = s * ROWS_PT
        ngrp_c = jnp.where(c == FASTC, (4 * GF) // nbuf, (4 * GS) // nbuf)
        pltpu.sync_copy(src_hbm.at[wid], src_v)
        pltpu.sync_copy(dst_hbm.at[wid], dst_v)
        pltpu.sync_copy(zeros_hbm.at[pl.ds(base, ROWS_PT)], acc.at[pl.ds(base, ROWS_PT)])
        plsc.subcore_barrier()

        def grp(g, carry):
            j = g * nbuf
            gs = [pltpu.async_copy(x_hbm.at[src_v.at[j + b]], bufs[b], sems[b])
                  for b in range(nbuf)]
            ss = []
            for b in range(nbuf):
                gs[b].wait()
                ss.append(pltpu.async_copy(bufs[b], acc.at[dst_v.at[j + b]],
                                           sems[nbuf + b], add=True))
            for d in ss:
                d.wait()
            return carry

        lax.fori_loop(0, ngrp_c, grp, 0)
        plsc.subcore_barrier()
        pltpu.sync_copy(acc.at[pl.ds(base, ROWS_PT)], out_hbm.at[c, pl.ds(base, ROWS_PT)])

    return body


def _make_agg1_body(nbuf):
    def body(xlo_hbm, xhi_hbm, src_hbm, dst_hbm, zeros_hbm, out_lo, out_hi,
             src_v, dst_v, *rest):
        bufs = rest[:nbuf]
        acc = rest[nbuf]
        sems = rest[nbuf + 1:]
        c = lax.axis_index("c")
        s = lax.axis_index("s")
        wid = s * NCORES + c
        base = s * ROWS_PT
        ngrp_c = jnp.where(c == FASTC, GF, GS)
        pltpu.sync_copy(src_hbm.at[wid], src_v)
        pltpu.sync_copy(dst_hbm.at[wid], dst_v)
        for x_hbm, out_hbm in ((xlo_hbm, out_lo), (xhi_hbm, out_hi)):
            pltpu.sync_copy(zeros_hbm.at[pl.ds(base, ROWS_PT)],
                            acc.at[pl.ds(base, ROWS_PT)])
            plsc.subcore_barrier()

            def grp(g, carry):
                j = g * nbuf
                gs = [pltpu.async_copy(x_hbm.at[src_v.at[j + b]], bufs[b], sems[b])
                      for b in range(nbuf)]
                ss = []
                for b in range(nbuf):
                    gs[b].wait()
                    ss.append(pltpu.async_copy(bufs[b], acc.at[dst_v.at[j + b]],
                                               sems[nbuf + b], add=True))
                for d in ss:
                    d.wait()
                return carry

            lax.fori_loop(0, ngrp_c, grp, 0)
            plsc.subcore_barrier()
            pltpu.sync_copy(acc.at[pl.ds(base, ROWS_PT)],
                            out_hbm.at[c, pl.ds(base, ROWS_PT)])
            plsc.subcore_barrier()

    return body


def _agg1_pass(xlo, xhi, src3, dst3, zeros, nbuf=4):
    nbatch = 4 * GF
    f = pl.kernel(
        _make_agg1_body(nbuf),
        mesh=_sc_mesh(),
        compiler_params=_SC_PARAMS,
        out_type=[
            jax.ShapeDtypeStruct((NCORES, NPAD, 64), jnp.float32),
            jax.ShapeDtypeStruct((NCORES, NPAD, 64), jnp.float32),
        ],
        scratch_types=[
            pltpu.VMEM((nbatch, BATCH), jnp.int32),
            pltpu.VMEM((nbatch, BATCH), jnp.int32),
        ] + [pltpu.VMEM((BATCH, 64), jnp.float32)] * nbuf + [
            pltpu.VMEM_SHARED((NPAD, 64), jnp.float32),
        ] + [pltpu.SemaphoreType.DMA] * (2 * nbuf),
    )
    return f(xlo, xhi, src3, dst3, zeros)


def _agg_pass(x, src3, dst3, zeros, width, batch, nbuf):
    nbatch = 4 * GF
    f = pl.kernel(
        _make_agg_body(width, batch, nbuf),
        mesh=_sc_mesh(),
        compiler_params=_SC_PARAMS,
        out_type=jax.ShapeDtypeStruct((NCORES, NPAD, width), jnp.float32),
        scratch_types=[
            pltpu.VMEM((nbatch, batch), jnp.int32),
            pltpu.VMEM((nbatch, batch), jnp.int32),
        ] + [pltpu.VMEM((batch, width), jnp.float32)] * nbuf + [
            pltpu.VMEM_SHARED((NPAD, width), jnp.float32),
        ] + [pltpu.SemaphoreType.DMA] * (2 * nbuf),
    )
    return f(x, src3, dst3, zeros)


def _tc_x1_body(ori_ref, struc_ref, odp_ref, w1o_ref, w1s_ref, lo_ref, hi_ref):
    odp = odp_ref[...]
    scale = lax.rsqrt(jnp.maximum(odp[0, :N, 0:1] + odp[1, :N, 0:1], 1.0))
    o = jnp.dot(ori_ref[...] * scale, w1o_ref[...], preferred_element_type=jnp.float32)
    st = jnp.dot(struc_ref[...] * scale, w1s_ref[...], preferred_element_type=jnp.float32)
    lo_ref[pl.ds(0, N), :] = o
    hi_ref[pl.ds(0, N), :] = st


def _tc_mid_body(aglo_ref, aghi_ref, odp_ref, idp_ref, b1o_ref, b1s_ref,
                 w2o_ref, w2s_ref, out_ref):
    odp = odp_ref[...]
    idp = idp_ref[...]
    odn = lax.rsqrt(jnp.maximum(odp[0, :N, 0:1] + odp[1, :N, 0:1], 1.0))
    idn = lax.rsqrt(jnp.maximum(idp[0, :N, 0:1] + idp[1, :N, 0:1], 1.0))
    aglo = aglo_ref[...]
    aghi = aghi_ref[...]
    h1o = jax.nn.relu((aglo[0, :N] + aglo[1, :N]) * idn + b1o_ref[...]) * odn
    h1s = jax.nn.relu((aghi[0, :N] + aghi[1, :N]) * idn + b1s_ref[...]) * odn
    x2o = jnp.dot(h1o, w2o_ref[...], preferred_element_type=jnp.float32)
    x2s = jnp.dot(h1s, w2s_ref[...], preferred_element_type=jnp.float32)
    out_ref[pl.ds(0, N), :] = jnp.concatenate([x2o, x2s], axis=1)


def _tc_mlp_body(aggp_ref, idp_ref, bcat_ref, m1_ref, mb1_ref, m2_ref, mb2_ref, out_ref):
    idp = idp_ref[...]
    idn = lax.rsqrt(jnp.maximum(idp[0, :N, 0:1] + idp[1, :N, 0:1], 1.0))
    aggp = aggp_ref[...]
    hc = (aggp[0, :N] + aggp[1, :N]) * idn + bcat_ref[...]
    h = jax.nn.relu(jnp.dot(hc, m1_ref[...], preferred_element_type=jnp.float32) + mb1_ref[...])
    out_ref[...] = jnp.dot(h, m2_ref[...], preferred_element_type=jnp.float32) + mb2_ref[...]


def _tc_call(body, out_shape, *args):
    return pl.pallas_call(body, out_shape=out_shape)(*args)


def kernel(ori_feat, struc_feat, edge_index, W1o, b1o, W2o, b2o,
           W1s, b1s, W2s, b2s, M1, mb1, M2, mb2):
    src = edge_index[0]
    dst = edge_index[1]
    pad = jnp.full((NW * EPT - E,), N, jnp.int32)
    srcp = jnp.concatenate([src, pad])
    dstp = jnp.concatenate([dst, pad])
    src3 = srcp.reshape(NW, NBATCH, BATCH)
    dst3 = dstp.reshape(NW, NBATCH, BATCH)

    def skew(ep):
        nbf, nbs = 4 * GF, 4 * GS
        fast = ep[:NSUB * nbf * BATCH].reshape(NSUB, nbf, BATCH)
        slow = jnp.full((NSUB, nbf, BATCH), N, jnp.int32)
        if nbs:
            tail = ep[NSUB * nbf * BATCH:].reshape(NSUB, nbs, BATCH)
            slow = slow.at[:, :nbs].set(tail)
        halves = (fast, slow) if FASTC == 0 else (slow, fast)
        return jnp.stack(halves, axis=1).reshape(NW, nbf, BATCH)

    srck = skew(srcp)
    dstk = skew(dstp)


    ones_deg = jnp.ones((BATCH, DEGW), jnp.float32)
    zeros_deg = jnp.zeros((NPAD, DEGW), jnp.float32)
    zeros64 = jnp.zeros((NPAD, 64), jnp.float32)
    zeros32 = jnp.zeros((NPAD, 32), jnp.float32)

    odp, idp = _degree_pass(src3, dst3, ones_deg, zeros_deg)

    xlo, xhi = _tc_call(_tc_x1_body,
                        [jax.ShapeDtypeStruct((NPAD, 64), jnp.float32),
                         jax.ShapeDtypeStruct((NPAD, 64), jnp.float32)],
                        ori_feat, struc_feat, odp, W1o, W1s)

    aglo, aghi = _agg1_pass(xlo, xhi, srck, dstk, zeros64)

    x2 = _tc_call(_tc_mid_body, jax.ShapeDtypeStruct((NPAD, 32), jnp.float32),
                  aglo, aghi, odp, idp, b1o.reshape(1, 64), b1s.reshape(1, 64), W2o, W2s)

    agg2 = _agg_pass(x2, srck, dstk, zeros32, 32, 128, 8)

    bcat = jnp.concatenate([b2o, b2s]).reshape(1, 32)
    out = _tc_call(_tc_mlp_body, jax.ShapeDtypeStruct((N, 16), jnp.float32),
                   agg2, idp, bcat, M1, mb1.reshape(1, 64), M2, mb2.reshape(1, 16))
    return out


# speedup vs baseline: 1.1361x; 1.1361x over previous
import jax
import jax.numpy as jnp
from jax import lax
from jax.experimental import pallas as pl
from jax.experimental.pallas import tpu as pltpu
from jax.experimental.pallas import tpu_sc as plsc

N = 10000
E = 320000
NCORES = 2
NSUB = 16
NW = NCORES * NSUB
BATCH = 128
EPT = 10240
NBATCH = EPT // BATCH
NBUF = 4
NGRP = NBATCH // NBUF
NPAD = 10240
ROWS_PT = NPAD // NSUB
DEGW = 8
FASTC = 0
GF = 30
GS = 10


def _sc_mesh():
    return plsc.VectorSubcoreMesh(core_axis_name="c", subcore_axis_name="s")


_SC_PARAMS = pltpu.CompilerParams(use_tc_tiling_on_sc=False)


def _deg_body(src_hbm, dst_hbm, ones_hbm, zeros_hbm, od_out, id_out,
              src_v, dst_v, ones_v, od_acc, id_acc, *sems):
    c = lax.axis_index("c")
    s = lax.axis_index("s")
    wid = s * NCORES + c
    base = s * ROWS_PT
    pltpu.sync_copy(ones_hbm, ones_v)
    pltpu.sync_copy(src_hbm.at[wid], src_v)
    pltpu.sync_copy(dst_hbm.at[wid], dst_v)
    pltpu.sync_copy(zeros_hbm.at[pl.ds(base, ROWS_PT)], od_acc.at[pl.ds(base, ROWS_PT)])
    pltpu.sync_copy(zeros_hbm.at[pl.ds(base, ROWS_PT)], id_acc.at[pl.ds(base, ROWS_PT)])
    plsc.subcore_barrier()

    def grp(g, carry):
        j = g * NBUF
        ds = []
        for b in range(NBUF):
            ds.append(pltpu.async_copy(ones_v, od_acc.at[src_v.at[j + b]],
                                       sems[b], add=True))
            ds.append(pltpu.async_copy(ones_v, id_acc.at[dst_v.at[j + b]],
                                       sems[NBUF + b], add=True))
        for d in ds:
            d.wait()
        return carry

    lax.fori_loop(0, NGRP, grp, 0)
    plsc.subcore_barrier()
    pltpu.sync_copy(od_acc.at[pl.ds(base, ROWS_PT)], od_out.at[c, pl.ds(base, ROWS_PT)])
    pltpu.sync_copy(id_acc.at[pl.ds(base, ROWS_PT)], id_out.at[c, pl.ds(base, ROWS_PT)])


def _degree_pass(src3, dst3, ones_deg, zeros_deg):
    f = pl.kernel(
        _deg_body,
        mesh=_sc_mesh(),
        compiler_params=_SC_PARAMS,
        out_type=[
            jax.ShapeDtypeStruct((NCORES, NPAD, DEGW), jnp.float32),
            jax.ShapeDtypeStruct((NCORES, NPAD, DEGW), jnp.float32),
        ],
        scratch_types=[
            pltpu.VMEM((NBATCH, BATCH), jnp.int32),
            pltpu.VMEM((NBATCH, BATCH), jnp.int32),
            pltpu.VMEM((BATCH, DEGW), jnp.float32),
            pltpu.VMEM_SHARED((NPAD, DEGW), jnp.float32),
            pltpu.VMEM_SHARED((NPAD, DEGW), jnp.float32),
        ] + [pltpu.SemaphoreType.DMA] * (2 * NBUF),
    )
    return f(src3, dst3, ones_deg, zeros_deg)


def _make_agg_body(width, batch, nbuf):
    def body(x_hbm, src_hbm, dst_hbm, zeros_hbm, out_hbm,
             src_v, dst_v, *rest):
        bufs = rest[:nbuf]
        acc = rest[nbuf]
        sems = rest[nbuf + 1:]
        c = lax.axis_index("c")
        s = lax.axis_index("s")
        wid = s * NCORES + c
        base = s * ROWS_PT
        ngrp_c = jnp.where(c == FASTC, (4 * GF) // nbuf, (4 * GS) // nbuf)
        pltpu.sync_copy(src_hbm.at[wid], src_v)
        pltpu.sync_copy(dst_hbm.at[wid], dst_v)
        pltpu.sync_copy(zeros_hbm.at[pl.ds(base, ROWS_PT)], acc.at[pl.ds(base, ROWS_PT)])
        plsc.subcore_barrier()

        def grp(g, carry):
            j = g * nbuf
            gs = [pltpu.async_copy(x_hbm.at[src_v.at[j + b]], bufs[b], sems[b])
                  for b in range(nbuf)]
            ss = []
            for b in range(nbuf):
                gs[b].wait()
                ss.append(pltpu.async_copy(bufs[b], acc.at[dst_v.at[j + b]],
                                           sems[nbuf + b], add=True))
            for d in ss:
                d.wait()
            return carry

        lax.fori_loop(0, ngrp_c, grp, 0)
        plsc.subcore_barrier()
        pltpu.sync_copy(acc.at[pl.ds(base, ROWS_PT)], out_hbm.at[c, pl.ds(base, ROWS_PT)])

    return body


def _make_agg1_body(nbuf):
    def body(xlo_hbm, xhi_hbm, src_hbm, dst_hbm, zeros_hbm, out_lo, out_hi,
             src_v, dst_v, *rest):
        bufs = rest[:nbuf]
        acc = rest[nbuf]
        sems = rest[nbuf + 1:]
        c = lax.axis_index("c")
        s = lax.axis_index("s")
        wid = s * NCORES + c
        base = s * ROWS_PT
        ngrp_c = jnp.where(c == FASTC, GF, GS)
        pltpu.sync_copy(src_hbm.at[wid], src_v)
        pltpu.sync_copy(dst_hbm.at[wid], dst_v)
        for x_hbm, out_hbm in ((xlo_hbm, out_lo), (xhi_hbm, out_hi)):
            pltpu.sync_copy(zeros_hbm.at[pl.ds(base, ROWS_PT)],
                            acc.at[pl.ds(base, ROWS_PT)])
            plsc.subcore_barrier()

            def grp(g, carry):
                j = g * nbuf
                gs = [pltpu.async_copy(x_hbm.at[src_v.at[j + b]], bufs[b], sems[b])
                      for b in range(nbuf)]
                ss = []
                for b in range(nbuf):
                    gs[b].wait()
                    ss.append(pltpu.async_copy(bufs[b], acc.at[dst_v.at[j + b]],
                                               sems[nbuf + b], add=True))
                for d in ss:
                    d.wait()
                return carry

            lax.fori_loop(0, ngrp_c, grp, 0)
            plsc.subcore_barrier()
            pltpu.sync_copy(acc.at[pl.ds(base, ROWS_PT)],
                            out_hbm.at[c, pl.ds(base, ROWS_PT)])
            plsc.subcore_barrier()

    return body


def _agg1_pass(xlo, xhi, src3, dst3, zeros, nbuf=4):
    nbatch = 4 * GF
    f = pl.kernel(
        _make_agg1_body(nbuf),
        mesh=_sc_mesh(),
        compiler_params=_SC_PARAMS,
        out_type=[
            jax.ShapeDtypeStruct((NCORES, NPAD, 64), jnp.float32),
            jax.ShapeDtypeStruct((NCORES, NPAD, 64), jnp.float32),
        ],
        scratch_types=[
            pltpu.VMEM((nbatch, BATCH), jnp.int32),
            pltpu.VMEM((nbatch, BATCH), jnp.int32),
        ] + [pltpu.VMEM((BATCH, 64), jnp.float32)] * nbuf + [
            pltpu.VMEM_SHARED((NPAD, 64), jnp.float32),
        ] + [pltpu.SemaphoreType.DMA] * (2 * nbuf),
    )
    return f(xlo, xhi, src3, dst3, zeros)


def _agg_pass(x, src3, dst3, zeros, width, batch, nbuf):
    nbatch = 4 * GF
    f = pl.kernel(
        _make_agg_body(width, batch, nbuf),
        mesh=_sc_mesh(),
        compiler_params=_SC_PARAMS,
        out_type=jax.ShapeDtypeStruct((NCORES, NPAD, width), jnp.float32),
        scratch_types=[
            pltpu.VMEM((nbatch, batch), jnp.int32),
            pltpu.VMEM((nbatch, batch), jnp.int32),
        ] + [pltpu.VMEM((batch, width), jnp.float32)] * nbuf + [
            pltpu.VMEM_SHARED((NPAD, width), jnp.float32),
        ] + [pltpu.SemaphoreType.DMA] * (2 * nbuf),
    )
    return f(x, src3, dst3, zeros)


def _tc_x1_body(ori_ref, struc_ref, odp_ref, w1o_ref, w1s_ref, lo_ref, hi_ref):
    odp = odp_ref[...]
    scale = lax.rsqrt(jnp.maximum(odp[0, :N, 0:1] + odp[1, :N, 0:1], 1.0))
    o = jnp.dot(ori_ref[...] * scale, w1o_ref[...], preferred_element_type=jnp.float32)
    st = jnp.dot(struc_ref[...] * scale, w1s_ref[...], preferred_element_type=jnp.float32)
    lo_ref[pl.ds(0, N), :] = o
    hi_ref[pl.ds(0, N), :] = st


def _tc_mid_body(aglo_ref, aghi_ref, odp_ref, idp_ref, b1o_ref, b1s_ref,
                 w2o_ref, w2s_ref, out_ref):
    odp = odp_ref[...]
    idp = idp_ref[...]
    odn = lax.rsqrt(jnp.maximum(odp[0, :N, 0:1] + odp[1, :N, 0:1], 1.0))
    idn = lax.rsqrt(jnp.maximum(idp[0, :N, 0:1] + idp[1, :N, 0:1], 1.0))
    aglo = aglo_ref[...]
    aghi = aghi_ref[...]
    h1o = jax.nn.relu((aglo[0, :N] + aglo[1, :N]) * idn + b1o_ref[...]) * odn
    h1s = jax.nn.relu((aghi[0, :N] + aghi[1, :N]) * idn + b1s_ref[...]) * odn
    x2o = jnp.dot(h1o, w2o_ref[...], preferred_element_type=jnp.float32)
    x2s = jnp.dot(h1s, w2s_ref[...], preferred_element_type=jnp.float32)
    out_ref[pl.ds(0, N), :] = jnp.concatenate([x2o, x2s], axis=1)


def _tc_mlp_body(aggp_ref, idp_ref, bcat_ref, m1_ref, mb1_ref, m2_ref, mb2_ref, out_ref):
    idp = idp_ref[...]
    idn = lax.rsqrt(jnp.maximum(idp[0, :N, 0:1] + idp[1, :N, 0:1], 1.0))
    aggp = aggp_ref[...]
    hc = (aggp[0, :N] + aggp[1, :N]) * idn + bcat_ref[...]
    h = jax.nn.relu(jnp.dot(hc, m1_ref[...], preferred_element_type=jnp.float32) + mb1_ref[...])
    out_ref[...] = jnp.dot(h, m2_ref[...], preferred_element_type=jnp.float32) + mb2_ref[...]


def _tc_call(body, out_shape, *args):
    return pl.pallas_call(body, out_shape=out_shape)(*args)


def kernel(ori_feat, struc_feat, edge_index, W1o, b1o, W2o, b2o,
           W1s, b1s, W2s, b2s, M1, mb1, M2, mb2):
    src = edge_index[0]
    dst = edge_index[1]
    pad = jnp.full((NW * EPT - E,), N, jnp.int32)
    srcp = jnp.concatenate([src, pad])
    dstp = jnp.concatenate([dst, pad])
    src3 = srcp.reshape(NW, NBATCH, BATCH)
    dst3 = dstp.reshape(NW, NBATCH, BATCH)

    def skew(ep):
        nbf, nbs = 4 * GF, 4 * GS
        fast = ep[:NSUB * nbf * BATCH].reshape(NSUB, nbf, BATCH)
        slow = jnp.full((NSUB, nbf, BATCH), N, jnp.int32)
        if nbs:
            tail = ep[NSUB * nbf * BATCH:].reshape(NSUB, nbs, BATCH)
            slow = slow.at[:, :nbs].set(tail)
        halves = (fast, slow) if FASTC == 0 else (slow, fast)
        return jnp.stack(halves, axis=1).reshape(NW, nbf, BATCH)

    srck = skew(srcp)
    dstk = skew(dstp)


    ones_deg = jnp.ones((BATCH, DEGW), jnp.float32)
    zeros_deg = jnp.zeros((NPAD, DEGW), jnp.float32)
    zeros64 = jnp.zeros((NPAD, 64), jnp.float32)
    zeros32 = jnp.zeros((NPAD, 32), jnp.float32)

    odp, idp = _degree_pass(src3, dst3, ones_deg, zeros_deg)

    xlo, xhi = _tc_call(_tc_x1_body,
                        [jax.ShapeDtypeStruct((NPAD, 64), jnp.float32),
                         jax.ShapeDtypeStruct((NPAD, 64), jnp.float32)],
                        ori_feat, struc_feat, odp, W1o, W1s)

    aglo, aghi = _agg1_pass(xlo, xhi, srck, dstk, zeros64)

    x2 = _tc_call(_tc_mid_body, jax.ShapeDtypeStruct((NPAD, 32), jnp.float32),
                  aglo, aghi, odp, idp, b1o.reshape(1, 64), b1s.reshape(1, 64), W2o, W2s)

    agg2 = _agg_pass(x2, srck, dstk, zeros32, 32, 128, 8)

    bcat = jnp.concatenate([b2o, b2s]).reshape(1, 32)
    out = _tc_call(_tc_mlp_body, jax.ShapeDtypeStruct((N, 16), jnp.float32),
                   agg2, idp, bcat, M1, mb1.reshape(1, 64), M2, mb2.reshape(1, 16))
    return out


# speedup vs baseline: 2.2652x; 1.9939x over previous
import jax
import jax.numpy as jnp
from jax import lax
from jax.experimental import pallas as pl
from jax.experimental.pallas import tpu as pltpu
from jax.experimental.pallas import tpu_sc as plsc

N = 10000
E = 320000
NCORES = 2
NSUB = 16
NW = NCORES * NSUB
BATCH = 128
EPT = 10240
NBATCH = EPT // BATCH
NBUF = 4
NGRP = NBATCH // NBUF
NPAD = 10240
ROWS_PT = NPAD // NSUB
DEGW = 8
FASTC = 0
GF = 20
GS = 20


def _sc_mesh():
    return plsc.VectorSubcoreMesh(core_axis_name="c", subcore_axis_name="s")


_SC_PARAMS = pltpu.CompilerParams(use_tc_tiling_on_sc=False)


def _deg_body(src_hbm, dst_hbm, ones_hbm, zeros_hbm, od_out, id_out,
              src_v, dst_v, ones_v, od_acc, id_acc, *sems):
    c = lax.axis_index("c")
    s = lax.axis_index("s")
    wid = s * NCORES + c
    base = s * ROWS_PT
    pltpu.sync_copy(ones_hbm, ones_v)
    pltpu.sync_copy(src_hbm.at[wid], src_v)
    pltpu.sync_copy(dst_hbm.at[wid], dst_v)
    pltpu.sync_copy(zeros_hbm.at[pl.ds(base, ROWS_PT)], od_acc.at[pl.ds(base, ROWS_PT)])
    pltpu.sync_copy(zeros_hbm.at[pl.ds(base, ROWS_PT)], id_acc.at[pl.ds(base, ROWS_PT)])
    plsc.subcore_barrier()

    def grp(g, carry):
        j = g * NBUF
        ds = []
        for b in range(NBUF):
            ds.append(pltpu.async_copy(ones_v, od_acc.at[src_v.at[j + b]],
                                       sems[b], add=True))
            ds.append(pltpu.async_copy(ones_v, id_acc.at[dst_v.at[j + b]],
                                       sems[NBUF + b], add=True))
        for d in ds:
            d.wait()
        return carry

    lax.fori_loop(0, NGRP, grp, 0)
    plsc.subcore_barrier()
    pltpu.sync_copy(od_acc.at[pl.ds(base, ROWS_PT)], od_out.at[c, pl.ds(base, ROWS_PT)])
    pltpu.sync_copy(id_acc.at[pl.ds(base, ROWS_PT)], id_out.at[c, pl.ds(base, ROWS_PT)])


def _degree_pass(src3, dst3, ones_deg, zeros_deg):
    f = pl.kernel(
        _deg_body,
        mesh=_sc_mesh(),
        compiler_params=_SC_PARAMS,
        out_type=[
            jax.ShapeDtypeStruct((NCORES, NPAD, DEGW), jnp.float32),
            jax.ShapeDtypeStruct((NCORES, NPAD, DEGW), jnp.float32),
        ],
        scratch_types=[
            pltpu.VMEM((NBATCH, BATCH), jnp.int32),
            pltpu.VMEM((NBATCH, BATCH), jnp.int32),
            pltpu.VMEM((BATCH, DEGW), jnp.float32),
            pltpu.VMEM_SHARED((NPAD, DEGW), jnp.float32),
            pltpu.VMEM_SHARED((NPAD, DEGW), jnp.float32),
        ] + [pltpu.SemaphoreType.DMA] * (2 * NBUF),
    )
    return f(src3, dst3, ones_deg, zeros_deg)


def _make_agg_body(width, batch, nbuf):
    def body(x_hbm, src_hbm, dst_hbm, zeros_hbm, out_hbm,
             src_v, dst_v, *rest):
        bufs = rest[:nbuf]
        acc = rest[nbuf]
        sems = rest[nbuf + 1:]
        c = lax.axis_index("c")
        s = lax.axis_index("s")
        wid = s * NCORES + c
        base = s * ROWS_PT
        ngrp_c = jnp.where(c == FASTC, (4 * GF) // nbuf, (4 * GS) // nbuf)
        pltpu.sync_copy(src_hbm.at[wid], src_v)
        pltpu.sync_copy(dst_hbm.at[wid], dst_v)
        pltpu.sync_copy(zeros_hbm.at[pl.ds(base, ROWS_PT)], acc.at[pl.ds(base, ROWS_PT)])
        plsc.subcore_barrier()

        def grp(g, carry):
            j = g * nbuf
            gs = [pltpu.async_copy(x_hbm.at[src_v.at[j + b]], bufs[b], sems[b])
                  for b in range(nbuf)]
            ss = []
            for b in range(nbuf):
                gs[b].wait()
                ss.append(pltpu.async_copy(bufs[b], acc.at[dst_v.at[j + b]],
                                           sems[nbuf + b], add=True))
            for d in ss:
                d.wait()
            return carry

        lax.fori_loop(0, ngrp_c, grp, 0)
        plsc.subcore_barrier()
        pltpu.sync_copy(acc.at[pl.ds(base, ROWS_PT)], out_hbm.at[c, pl.ds(base, ROWS_PT)])

    return body


def _make_agg1_body(nbuf):
    def body(xlo_hbm, xhi_hbm, src_hbm, dst_hbm, zeros_hbm, out_lo, out_hi,
             src_v, dst_v, *rest):
        bufs = rest[:nbuf]
        acc = rest[nbuf]
        sems = rest[nbuf + 1:]
        c = lax.axis_index("c")
        s = lax.axis_index("s")
        wid = s * NCORES + c
        base = s * ROWS_PT
        ngrp_c = jnp.where(c == FASTC, GF, GS)
        pltpu.sync_copy(src_hbm.at[wid], src_v)
        pltpu.sync_copy(dst_hbm.at[wid], dst_v)
        for x_hbm, out_hbm in ((xlo_hbm, out_lo), (xhi_hbm, out_hi)):
            pltpu.sync_copy(zeros_hbm.at[pl.ds(base, ROWS_PT)],
                            acc.at[pl.ds(base, ROWS_PT)])
            plsc.subcore_barrier()

            def grp(g, carry):
                j = g * nbuf
                gs = [pltpu.async_copy(x_hbm.at[src_v.at[j + b]], bufs[b], sems[b])
                      for b in range(nbuf)]
                ss = []
                for b in range(nbuf):
                    gs[b].wait()
                    ss.append(pltpu.async_copy(bufs[b], acc.at[dst_v.at[j + b]],
                                               sems[nbuf + b], add=True))
                for d in ss:
                    d.wait()
                return carry

            lax.fori_loop(0, ngrp_c, grp, 0)
            plsc.subcore_barrier()
            pltpu.sync_copy(acc.at[pl.ds(base, ROWS_PT)],
                            out_hbm.at[c, pl.ds(base, ROWS_PT)])
            plsc.subcore_barrier()

    return body


def _agg1_pass(xlo, xhi, src3, dst3, zeros, nbuf=4):
    nbatch = 4 * GF
    f = pl.kernel(
        _make_agg1_body(nbuf),
        mesh=_sc_mesh(),
        compiler_params=_SC_PARAMS,
        out_type=[
            jax.ShapeDtypeStruct((NCORES, NPAD, 64), jnp.float32),
            jax.ShapeDtypeStruct((NCORES, NPAD, 64), jnp.float32),
        ],
        scratch_types=[
            pltpu.VMEM((nbatch, BATCH), jnp.int32),
            pltpu.VMEM((nbatch, BATCH), jnp.int32),
        ] + [pltpu.VMEM((BATCH, 64), jnp.float32)] * nbuf + [
            pltpu.VMEM_SHARED((NPAD, 64), jnp.float32),
        ] + [pltpu.SemaphoreType.DMA] * (2 * nbuf),
    )
    return f(xlo, xhi, src3, dst3, zeros)


def _agg_pass(x, src3, dst3, zeros, width, batch, nbuf):
    nbatch = 4 * GF
    f = pl.kernel(
        _make_agg_body(width, batch, nbuf),
        mesh=_sc_mesh(),
        compiler_params=_SC_PARAMS,
        out_type=jax.ShapeDtypeStruct((NCORES, NPAD, width), jnp.float32),
        scratch_types=[
            pltpu.VMEM((nbatch, batch), jnp.int32),
            pltpu.VMEM((nbatch, batch), jnp.int32),
        ] + [pltpu.VMEM((batch, width), jnp.float32)] * nbuf + [
            pltpu.VMEM_SHARED((NPAD, width), jnp.float32),
        ] + [pltpu.SemaphoreType.DMA] * (2 * nbuf),
    )
    return f(x, src3, dst3, zeros)


def _tc_x1_body(ori_ref, struc_ref, odp_ref, w1o_ref, w1s_ref, lo_ref, hi_ref):
    odp = odp_ref[...]
    scale = lax.rsqrt(jnp.maximum(odp[0, :N, 0:1] + odp[1, :N, 0:1], 1.0))
    o = jnp.dot(ori_ref[...] * scale, w1o_ref[...], preferred_element_type=jnp.float32)
    st = jnp.dot(struc_ref[...] * scale, w1s_ref[...], preferred_element_type=jnp.float32)
    lo_ref[pl.ds(0, N), :] = o
    hi_ref[pl.ds(0, N), :] = st


def _tc_mid_body(aglo_ref, aghi_ref, odp_ref, idp_ref, b1o_ref, b1s_ref,
                 w2o_ref, w2s_ref, out_ref):
    odp = odp_ref[...]
    idp = idp_ref[...]
    odn = lax.rsqrt(jnp.maximum(odp[0, :N, 0:1] + odp[1, :N, 0:1], 1.0))
    idn = lax.rsqrt(jnp.maximum(idp[0, :N, 0:1] + idp[1, :N, 0:1], 1.0))
    aglo = aglo_ref[...]
    aghi = aghi_ref[...]
    h1o = jax.nn.relu((aglo[0, :N] + aglo[1, :N]) * idn + b1o_ref[...]) * odn
    h1s = jax.nn.relu((aghi[0, :N] + aghi[1, :N]) * idn + b1s_ref[...]) * odn
    x2o = jnp.dot(h1o, w2o_ref[...], preferred_element_type=jnp.float32)
    x2s = jnp.dot(h1s, w2s_ref[...], preferred_element_type=jnp.float32)
    out_ref[pl.ds(0, N), :] = jnp.concatenate([x2o, x2s], axis=1)


def _tc_mlp_body(aggp_ref, idp_ref, bcat_ref, m1_ref, mb1_ref, m2_ref, mb2_ref, out_ref):
    idp = idp_ref[...]
    idn = lax.rsqrt(jnp.maximum(idp[0, :N, 0:1] + idp[1, :N, 0:1], 1.0))
    aggp = aggp_ref[...]
    hc = (aggp[0, :N] + aggp[1, :N]) * idn + bcat_ref[...]
    h = jax.nn.relu(jnp.dot(hc, m1_ref[...], preferred_element_type=jnp.float32) + mb1_ref[...])
    out_ref[...] = jnp.dot(h, m2_ref[...], preferred_element_type=jnp.float32) + mb2_ref[...]


def _tc_call(body, out_shape, *args):
    return pl.pallas_call(body, out_shape=out_shape)(*args)


def kernel(ori_feat, struc_feat, edge_index, W1o, b1o, W2o, b2o,
           W1s, b1s, W2s, b2s, M1, mb1, M2, mb2):
    src = edge_index[0]
    dst = edge_index[1]
    pad = N + (jnp.arange(NW * EPT - E, dtype=jnp.int32) % (NPAD - N))
    srcp = jnp.concatenate([src, pad])
    dstp = jnp.concatenate([dst, pad])
    src3 = srcp.reshape(NW, NBATCH, BATCH)
    dst3 = dstp.reshape(NW, NBATCH, BATCH)

    def skew(ep):
        nbf, nbs = 4 * GF, 4 * GS
        fast = ep[:NSUB * nbf * BATCH].reshape(NSUB, nbf, BATCH)
        slow = N + (jnp.arange(NSUB * nbf * BATCH, dtype=jnp.int32)
                    % (NPAD - N)).reshape(NSUB, nbf, BATCH)
        if nbs:
            tail = ep[NSUB * nbf * BATCH:].reshape(NSUB, nbs, BATCH)
            slow = slow.at[:, :nbs].set(tail)
        halves = (fast, slow) if FASTC == 0 else (slow, fast)
        return jnp.stack(halves, axis=1).reshape(NW, nbf, BATCH)

    srck = skew(srcp)
    dstk = skew(dstp)


    ones_deg = jnp.ones((BATCH, DEGW), jnp.float32)
    zeros_deg = jnp.zeros((NPAD, DEGW), jnp.float32)
    zeros64 = jnp.zeros((NPAD, 64), jnp.float32)
    zeros32 = jnp.zeros((NPAD, 32), jnp.float32)

    odp, idp = _degree_pass(src3, dst3, ones_deg, zeros_deg)

    xlo, xhi = _tc_call(_tc_x1_body,
                        [jax.ShapeDtypeStruct((NPAD, 64), jnp.float32),
                         jax.ShapeDtypeStruct((NPAD, 64), jnp.float32)],
                        ori_feat, struc_feat, odp, W1o, W1s)

    aglo, aghi = _agg1_pass(xlo, xhi, srck, dstk, zeros64)

    x2 = _tc_call(_tc_mid_body, jax.ShapeDtypeStruct((NPAD, 32), jnp.float32),
                  aglo, aghi, odp, idp, b1o.reshape(1, 64), b1s.reshape(1, 64), W2o, W2s)

    agg2 = _agg_pass(x2, srck, dstk, zeros32, 32, 128, 8)

    bcat = jnp.concatenate([b2o, b2s]).reshape(1, 32)
    out = _tc_call(_tc_mlp_body, jax.ShapeDtypeStruct((N, 16), jnp.float32),
                   agg2, idp, bcat, M1, mb1.reshape(1, 64), M2, mb2.reshape(1, 16))
    return out
